# baseline (device time: 214631 ns/iter reference)
import jax
import jax.numpy as jnp
from jax import lax
from jax.experimental import pallas as pl
from jax.experimental.pallas import tpu as pltpu

N_DEV = 16


def kernel(x, w_mat):
    m, k_shard = x.shape
    _, n = w_mat.shape
    m_blk = m // N_DEV

    def body(x_ref, w_ref, out_ref, wb_ref, comm_ref, send_sems, recv_sems):
        d = lax.axis_index("i")
        left = lax.rem(d + N_DEV - 1, N_DEV)
        right = lax.rem(d + 1, N_DEV)

        barrier_sem = pltpu.get_barrier_semaphore()
        pl.semaphore_signal(barrier_sem, inc=1, device_id=(left,),
                            device_id_type=pl.DeviceIdType.MESH)
        pl.semaphore_signal(barrier_sem, inc=1, device_id=(right,),
                            device_id_type=pl.DeviceIdType.MESH)
        pl.semaphore_wait(barrier_sem, 2)

        wb_ref[...] = w_ref[...].astype(jnp.bfloat16)

        def partial(c):
            xa = x_ref[pl.ds(c * m_blk, m_blk), :].astype(jnp.bfloat16)
            return jnp.dot(xa, wb_ref[...], preferred_element_type=jnp.float32)

        comm_ref[0] = partial(left).astype(jnp.bfloat16)

        for s in range(N_DEV - 1):
            rdma = pltpu.make_async_remote_copy(
                src_ref=comm_ref.at[s],
                dst_ref=comm_ref.at[s + 1],
                send_sem=send_sems.at[s],
                recv_sem=recv_sems.at[s],
                device_id=(right,),
                device_id_type=pl.DeviceIdType.MESH,
            )
            rdma.start()
            c = lax.rem(d + 2 * N_DEV - 2 - s, N_DEV)
            p = partial(c)
            rdma.wait()
            acc = comm_ref[s + 1].astype(jnp.float32) + p
            if s < N_DEV - 2:
                comm_ref[s + 1] = acc.astype(jnp.bfloat16)
            else:
                out_ref[...] = acc * jax.nn.sigmoid(acc)

        def exit_barrier(sem):
            pl.semaphore_signal(sem, inc=1, device_id=(left,),
                                device_id_type=pl.DeviceIdType.MESH)
            pl.semaphore_signal(sem, inc=1, device_id=(right,),
                                device_id_type=pl.DeviceIdType.MESH)
            pl.semaphore_wait(sem, 2)

        pl.run_scoped(exit_barrier, pltpu.SemaphoreType.REGULAR)

    return pl.pallas_call(
        body,
        out_shape=jax.ShapeDtypeStruct((m_blk, n), jnp.float32),
        in_specs=[
            pl.BlockSpec(memory_space=pltpu.VMEM),
            pl.BlockSpec(memory_space=pltpu.VMEM),
        ],
        out_specs=pl.BlockSpec(memory_space=pltpu.VMEM),
        scratch_shapes=[
            pltpu.VMEM((k_shard, n), jnp.bfloat16),
            pltpu.VMEM((N_DEV, m_blk, n), jnp.bfloat16),
            pltpu.SemaphoreType.DMA((N_DEV - 1,)),
            pltpu.SemaphoreType.DMA((N_DEV - 1,)),
        ],
        compiler_params=pltpu.CompilerParams(collective_id=0),
    )(x, w_mat)


# device time: 148039 ns/iter; 1.4498x vs baseline; 1.4498x over previous
import jax
import jax.numpy as jnp
from jax import lax
from jax.experimental import pallas as pl
from jax.experimental.pallas import tpu as pltpu

N_DEV = 16


def kernel(x, w_mat):
    m, k_shard = x.shape
    _, n = w_mat.shape
    m_blk = m // N_DEV
    n_half = n // 2

    def body(x_ref, w_ref, out_ref, wb_ref, comm_r, comm_l,
             send_r, recv_r, send_l, recv_l):
        d = lax.axis_index("i")
        left = lax.rem(d + N_DEV - 1, N_DEV)
        right = lax.rem(d + 1, N_DEV)

        barrier_sem = pltpu.get_barrier_semaphore()
        pl.semaphore_signal(barrier_sem, inc=1, device_id=(left,),
                            device_id_type=pl.DeviceIdType.MESH)
        pl.semaphore_signal(barrier_sem, inc=1, device_id=(right,),
                            device_id_type=pl.DeviceIdType.MESH)
        pl.semaphore_wait(barrier_sem, 2)

        wb_ref[...] = w_ref[...].astype(jnp.bfloat16)

        def partial(c, half):
            xa = x_ref[pl.ds(c * m_blk, m_blk), :].astype(jnp.bfloat16)
            wh = wb_ref[:, half * n_half:(half + 1) * n_half]
            return jnp.dot(xa, wh, preferred_element_type=jnp.float32)

        comm_r[0] = partial(left, 0).astype(jnp.bfloat16)
        comm_l[0] = partial(right, 1).astype(jnp.bfloat16)

        for s in range(N_DEV - 1):
            rdma_r = pltpu.make_async_remote_copy(
                src_ref=comm_r.at[s], dst_ref=comm_r.at[s + 1],
                send_sem=send_r.at[s], recv_sem=recv_r.at[s],
                device_id=(right,), device_id_type=pl.DeviceIdType.MESH,
            )
            rdma_l = pltpu.make_async_remote_copy(
                src_ref=comm_l.at[s], dst_ref=comm_l.at[s + 1],
                send_sem=send_l.at[s], recv_sem=recv_l.at[s],
                device_id=(left,), device_id_type=pl.DeviceIdType.MESH,
            )
            rdma_r.start()
            rdma_l.start()
            c_r = lax.rem(d + 2 * N_DEV - 2 - s, N_DEV)
            c_l = lax.rem(d + 2 + s, N_DEV)
            p_r = partial(c_r, 0)
            p_l = partial(c_l, 1)
            rdma_r.wait()
            acc_r = comm_r[s + 1].astype(jnp.float32) + p_r
            if s < N_DEV - 2:
                comm_r[s + 1] = acc_r.astype(jnp.bfloat16)
            rdma_l.wait()
            acc_l = comm_l[s + 1].astype(jnp.float32) + p_l
            if s < N_DEV - 2:
                comm_l[s + 1] = acc_l.astype(jnp.bfloat16)
            else:
                out_ref[:, 0:n_half] = acc_r * jax.nn.sigmoid(acc_r)
                out_ref[:, n_half:n] = acc_l * jax.nn.sigmoid(acc_l)

        def exit_barrier(sem):
            pl.semaphore_signal(sem, inc=1, device_id=(left,),
                                device_id_type=pl.DeviceIdType.MESH)
            pl.semaphore_signal(sem, inc=1, device_id=(right,),
                                device_id_type=pl.DeviceIdType.MESH)
            pl.semaphore_wait(sem, 2)

        pl.run_scoped(exit_barrier, pltpu.SemaphoreType.REGULAR)

    return pl.pallas_call(
        body,
        out_shape=jax.ShapeDtypeStruct((m_blk, n), jnp.float32),
        in_specs=[
            pl.BlockSpec(memory_space=pltpu.VMEM),
            pl.BlockSpec(memory_space=pltpu.VMEM),
        ],
        out_specs=pl.BlockSpec(memory_space=pltpu.VMEM),
        scratch_shapes=[
            pltpu.VMEM((k_shard, n), jnp.bfloat16),
            pltpu.VMEM((N_DEV, m_blk, n_half), jnp.bfloat16),
            pltpu.VMEM((N_DEV, m_blk, n_half), jnp.bfloat16),
            pltpu.SemaphoreType.DMA((N_DEV - 1,)),
            pltpu.SemaphoreType.DMA((N_DEV - 1,)),
            pltpu.SemaphoreType.DMA((N_DEV - 1,)),
            pltpu.SemaphoreType.DMA((N_DEV - 1,)),
        ],
        compiler_params=pltpu.CompilerParams(collective_id=0),
    )(x, w_mat)


# device time: 124450 ns/iter; 1.7246x vs baseline; 1.1895x over previous
import jax
import jax.numpy as jnp
from jax import lax
from jax.experimental import pallas as pl
from jax.experimental.pallas import tpu as pltpu

N_DEV = 16

RING = (0, 4, 8, 12, 15, 11, 7, 3, 2, 6, 10, 14, 13, 9, 5, 1)
POS = tuple(RING.index(i) for i in range(N_DEV))


def _lut(table, idx):
    out = jnp.int32(table[0])
    for j in range(1, len(table)):
        out = jnp.where(idx == j, jnp.int32(table[j]), out)
    return out


def kernel(x, w_mat):
    m, k_shard = x.shape
    _, n = w_mat.shape
    m_blk = m // N_DEV
    n_half = n // 2

    def body(x_ref, w_ref, out_ref, wb_ref, comm_r, comm_l,
             send_r, recv_r, send_l, recv_l):
        d = lax.axis_index("i")
        r_pos = _lut(POS, d)
        right = _lut(RING, lax.rem(r_pos + 1, N_DEV))
        left = _lut(RING, lax.rem(r_pos + N_DEV - 1, N_DEV))

        barrier_sem = pltpu.get_barrier_semaphore()
        pl.semaphore_signal(barrier_sem, inc=1, device_id=(left,),
                            device_id_type=pl.DeviceIdType.MESH)
        pl.semaphore_signal(barrier_sem, inc=1, device_id=(right,),
                            device_id_type=pl.DeviceIdType.MESH)
        pl.semaphore_wait(barrier_sem, 2)

        wb_ref[...] = w_ref[...].astype(jnp.bfloat16)

        def partial(c, half):
            xa = x_ref[pl.ds(c * m_blk, m_blk), :].astype(jnp.bfloat16)
            wh = wb_ref[:, half * n_half:(half + 1) * n_half]
            return jnp.dot(xa, wh, preferred_element_type=jnp.float32)

        def chunk_r(s):
            return _lut(RING, lax.rem(r_pos + 2 * N_DEV - 2 - s, N_DEV))

        def chunk_l(s):
            return _lut(RING, lax.rem(r_pos + 2 + s, N_DEV))

        def mk(comm, sends, recvs, s, dst):
            return pltpu.make_async_remote_copy(
                src_ref=comm.at[s], dst_ref=comm.at[s + 1],
                send_sem=sends.at[s], recv_sem=recvs.at[s],
                device_id=(dst,), device_id_type=pl.DeviceIdType.MESH,
            )

        comm_r[0] = partial(left, 0).astype(jnp.bfloat16)
        mk(comm_r, send_r, recv_r, 0, right).start()
        comm_l[0] = partial(right, 1).astype(jnp.bfloat16)
        mk(comm_l, send_l, recv_l, 0, left).start()

        for s in range(N_DEV - 1):
            last = s == N_DEV - 2
            p_r = partial(chunk_r(s), 0)
            mk(comm_r, send_r, recv_r, s, right).wait()
            acc_r = comm_r[s + 1].astype(jnp.float32) + p_r
            if not last:
                comm_r[s + 1] = acc_r.astype(jnp.bfloat16)
                mk(comm_r, send_r, recv_r, s + 1, right).start()
            p_l = partial(chunk_l(s), 1)
            mk(comm_l, send_l, recv_l, s, left).wait()
            acc_l = comm_l[s + 1].astype(jnp.float32) + p_l
            if not last:
                comm_l[s + 1] = acc_l.astype(jnp.bfloat16)
                mk(comm_l, send_l, recv_l, s + 1, left).start()
            else:
                out_ref[:, 0:n_half] = acc_r * jax.nn.sigmoid(acc_r)
                out_ref[:, n_half:n] = acc_l * jax.nn.sigmoid(acc_l)

        def exit_barrier(sem):
            pl.semaphore_signal(sem, inc=1, device_id=(left,),
                                device_id_type=pl.DeviceIdType.MESH)
            pl.semaphore_signal(sem, inc=1, device_id=(right,),
                                device_id_type=pl.DeviceIdType.MESH)
            pl.semaphore_wait(sem, 2)

        pl.run_scoped(exit_barrier, pltpu.SemaphoreType.REGULAR)

    return pl.pallas_call(
        body,
        out_shape=jax.ShapeDtypeStruct((m_blk, n), jnp.float32),
        in_specs=[
            pl.BlockSpec(memory_space=pltpu.VMEM),
            pl.BlockSpec(memory_space=pltpu.VMEM),
        ],
        out_specs=pl.BlockSpec(memory_space=pltpu.VMEM),
        scratch_shapes=[
            pltpu.VMEM((k_shard, n), jnp.bfloat16),
            pltpu.VMEM((N_DEV, m_blk, n_half), jnp.bfloat16),
            pltpu.VMEM((N_DEV, m_blk, n_half), jnp.bfloat16),
            pltpu.SemaphoreType.DMA((N_DEV - 1,)),
            pltpu.SemaphoreType.DMA((N_DEV - 1,)),
            pltpu.SemaphoreType.DMA((N_DEV - 1,)),
            pltpu.SemaphoreType.DMA((N_DEV - 1,)),
        ],
        compiler_params=pltpu.CompilerParams(collective_id=0),
    )(x, w_mat)


# device time: 98487 ns/iter; 2.1793x vs baseline; 1.2636x over previous
import jax
import jax.numpy as jnp
from jax import lax
from jax.experimental import pallas as pl
from jax.experimental.pallas import tpu as pltpu

N_DEV = 16

RING = (0, 4, 8, 12, 15, 11, 7, 3, 2, 6, 10, 14, 13, 9, 5, 1)
POS = tuple(RING.index(i) for i in range(N_DEV))


def _lut(table, idx):
    out = jnp.int32(table[0])
    for j in range(1, len(table)):
        out = jnp.where(idx == j, jnp.int32(table[j]), out)
    return out


def kernel(x, w_mat):
    m, k_shard = x.shape
    _, n = w_mat.shape
    m_blk = m // N_DEV
    n_half = n // 2
    n_sub = n_half // 2

    def body(x_ref, w_ref, out_ref, wb_ref,
             comm_ra, comm_rb, comm_la, comm_lb,
             send_ra, recv_ra, send_rb, recv_rb,
             send_la, recv_la, send_lb, recv_lb):
        d = lax.axis_index("i")
        r_pos = _lut(POS, d)
        right = _lut(RING, lax.rem(r_pos + 1, N_DEV))
        left = _lut(RING, lax.rem(r_pos + N_DEV - 1, N_DEV))

        barrier_sem = pltpu.get_barrier_semaphore()
        pl.semaphore_signal(barrier_sem, inc=1, device_id=(left,),
                            device_id_type=pl.DeviceIdType.MESH)
        pl.semaphore_signal(barrier_sem, inc=1, device_id=(right,),
                            device_id_type=pl.DeviceIdType.MESH)
        pl.semaphore_wait(barrier_sem, 2)

        wb_ref[...] = w_ref[...].astype(jnp.bfloat16)

        def partial(c, half):
            xa = x_ref[pl.ds(c * m_blk, m_blk), :].astype(jnp.bfloat16)
            wh = wb_ref[:, half * n_half:(half + 1) * n_half]
            return jnp.dot(xa, wh, preferred_element_type=jnp.float32)

        def chunk_r(s):
            return _lut(RING, lax.rem(r_pos + 2 * N_DEV - 2 - s, N_DEV))

        def chunk_l(s):
            return _lut(RING, lax.rem(r_pos + 2 + s, N_DEV))

        def mk(comm, sends, recvs, s, dst):
            return pltpu.make_async_remote_copy(
                src_ref=comm.at[s], dst_ref=comm.at[s + 1],
                send_sem=sends.at[s], recv_sem=recvs.at[s],
                device_id=(dst,), device_id_type=pl.DeviceIdType.MESH,
            )

        streams = (
            (comm_ra, send_ra, recv_ra, right, 0, 0),
            (comm_rb, send_rb, recv_rb, right, 0, n_sub),
            (comm_la, send_la, recv_la, left, 1, 0),
            (comm_lb, send_lb, recv_lb, left, 1, n_sub),
        )

        p0_r = partial(left, 0).astype(jnp.bfloat16)
        comm_ra[0] = p0_r[:, 0:n_sub]
        mk(comm_ra, send_ra, recv_ra, 0, right).start()
        comm_rb[0] = p0_r[:, n_sub:n_half]
        mk(comm_rb, send_rb, recv_rb, 0, right).start()
        p0_l = partial(right, 1).astype(jnp.bfloat16)
        comm_la[0] = p0_l[:, 0:n_sub]
        mk(comm_la, send_la, recv_la, 0, left).start()
        comm_lb[0] = p0_l[:, n_sub:n_half]
        mk(comm_lb, send_lb, recv_lb, 0, left).start()

        for s in range(N_DEV - 1):
            last = s == N_DEV - 2
            p_r = partial(chunk_r(s), 0)
            p_l = partial(chunk_l(s), 1)
            accs = []
            for comm, sends, recvs, dst, half, off in streams:
                p = p_r if half == 0 else p_l
                mk(comm, sends, recvs, s, dst).wait()
                acc = comm[s + 1].astype(jnp.float32) + p[:, off:off + n_sub]
                if not last:
                    comm[s + 1] = acc.astype(jnp.bfloat16)
                    mk(comm, sends, recvs, s + 1, dst).start()
                else:
                    accs.append((half * n_half + off, acc))
            if last:
                for col, acc in accs:
                    out_ref[:, col:col + n_sub] = acc * jax.nn.sigmoid(acc)

        def exit_barrier(sem):
            pl.semaphore_signal(sem, inc=1, device_id=(left,),
                                device_id_type=pl.DeviceIdType.MESH)
            pl.semaphore_signal(sem, inc=1, device_id=(right,),
                                device_id_type=pl.DeviceIdType.MESH)
            pl.semaphore_wait(sem, 2)

        pl.run_scoped(exit_barrier, pltpu.SemaphoreType.REGULAR)

    sub = pltpu.VMEM((N_DEV, m_blk, n_sub), jnp.bfloat16)
    sems = pltpu.SemaphoreType.DMA((N_DEV - 1,))
    return pl.pallas_call(
        body,
        out_shape=jax.ShapeDtypeStruct((m_blk, n), jnp.float32),
        in_specs=[
            pl.BlockSpec(memory_space=pltpu.VMEM),
            pl.BlockSpec(memory_space=pltpu.VMEM),
        ],
        out_specs=pl.BlockSpec(memory_space=pltpu.VMEM),
        scratch_shapes=[
            pltpu.VMEM((k_shard, n), jnp.bfloat16),
            sub, sub, sub, sub,
            sems, sems, sems, sems, sems, sems, sems, sems,
        ],
        compiler_params=pltpu.CompilerParams(collective_id=0),
    )(x, w_mat)


# device time: 98064 ns/iter; 2.1887x vs baseline; 1.0043x over previous
import jax
import jax.numpy as jnp
from jax import lax
from jax.experimental import pallas as pl
from jax.experimental.pallas import tpu as pltpu

N_DEV = 16

RING = (0, 4, 8, 12, 15, 11, 7, 3, 2, 6, 10, 14, 13, 9, 5, 1)
POS = tuple(RING.index(i) for i in range(N_DEV))


def _lut(table, idx):
    out = jnp.int32(table[0])
    for j in range(1, len(table)):
        out = jnp.where(idx == j, jnp.int32(table[j]), out)
    return out


def kernel(x, w_mat):
    m, k_shard = x.shape
    _, n = w_mat.shape
    m_blk = m // N_DEV
    n_half = n // 2
    n_sub = n_half // 2

    def body(x_ref, w_ref, out_ref, wb_ref,
             comm_ra, comm_rb, comm_la, comm_lb,
             send_ra, recv_ra, send_rb, recv_rb,
             send_la, recv_la, send_lb, recv_lb):
        d = lax.axis_index("i")
        r_pos = _lut(POS, d)
        right = _lut(RING, lax.rem(r_pos + 1, N_DEV))
        left = _lut(RING, lax.rem(r_pos + N_DEV - 1, N_DEV))

        barrier_sem = pltpu.get_barrier_semaphore()
        pl.semaphore_signal(barrier_sem, inc=1, device_id=(left,),
                            device_id_type=pl.DeviceIdType.MESH)
        pl.semaphore_signal(barrier_sem, inc=1, device_id=(right,),
                            device_id_type=pl.DeviceIdType.MESH)
        pl.semaphore_wait(barrier_sem, 2)

        wb_ref[...] = w_ref[...].astype(jnp.bfloat16)

        def partial(c, half):
            xa = x_ref[pl.ds(c * m_blk, m_blk), :].astype(jnp.bfloat16)
            wh = wb_ref[:, half * n_half:(half + 1) * n_half]
            return jnp.dot(xa, wh, preferred_element_type=jnp.float32)

        def chunk_r(s):
            return _lut(RING, lax.rem(r_pos + 2 * N_DEV - 2 - s, N_DEV))

        def chunk_l(s):
            return _lut(RING, lax.rem(r_pos + 2 + s, N_DEV))

        def mk(comm, sends, recvs, s, dst):
            return pltpu.make_async_remote_copy(
                src_ref=comm.at[s], dst_ref=comm.at[s + 1],
                send_sem=sends.at[s], recv_sem=recvs.at[s],
                device_id=(dst,), device_id_type=pl.DeviceIdType.MESH,
            )

        streams = (
            (comm_ra, send_ra, recv_ra, right, 0, 0),
            (comm_rb, send_rb, recv_rb, right, 0, n_sub),
            (comm_la, send_la, recv_la, left, 1, 0),
            (comm_lb, send_lb, recv_lb, left, 1, n_sub),
        )

        def sub_partial(c, half, off):
            xa = x_ref[pl.ds(c * m_blk, m_blk), :].astype(jnp.bfloat16)
            lo = half * n_half + off
            return jnp.dot(xa, wb_ref[:, lo:lo + n_sub],
                           preferred_element_type=jnp.float32)

        for comm, sends, recvs, dst, half, off in streams:
            c0 = left if half == 0 else right
            comm[0] = sub_partial(c0, half, off).astype(jnp.bfloat16)
            mk(comm, sends, recvs, 0, dst).start()

        for s in range(N_DEV - 1):
            last = s == N_DEV - 2
            p_r = partial(chunk_r(s), 0)
            p_l = partial(chunk_l(s), 1)
            for comm, sends, recvs, dst, half, off in streams:
                p = p_r if half == 0 else p_l
                mk(comm, sends, recvs, s, dst).wait()
                acc = comm[s + 1].astype(jnp.float32) + p[:, off:off + n_sub]
                if not last:
                    comm[s + 1] = acc.astype(jnp.bfloat16)
                    mk(comm, sends, recvs, s + 1, dst).start()
                else:
                    col = half * n_half + off
                    out_ref[:, col:col + n_sub] = acc * jax.nn.sigmoid(acc)

        def exit_barrier(sem):
            pl.semaphore_signal(sem, inc=1, device_id=(left,),
                                device_id_type=pl.DeviceIdType.MESH)
            pl.semaphore_signal(sem, inc=1, device_id=(right,),
                                device_id_type=pl.DeviceIdType.MESH)
            pl.semaphore_wait(sem, 2)

        pl.run_scoped(exit_barrier, pltpu.SemaphoreType.REGULAR)

    sub = pltpu.VMEM((N_DEV, m_blk, n_sub), jnp.bfloat16)
    sems = pltpu.SemaphoreType.DMA((N_DEV - 1,))
    return pl.pallas_call(
        body,
        out_shape=jax.ShapeDtypeStruct((m_blk, n), jnp.float32),
        in_specs=[
            pl.BlockSpec(memory_space=pltpu.VMEM),
            pl.BlockSpec(memory_space=pltpu.VMEM),
        ],
        out_specs=pl.BlockSpec(memory_space=pltpu.VMEM),
        scratch_shapes=[
            pltpu.VMEM((k_shard, n), jnp.bfloat16),
            sub, sub, sub, sub,
            sems, sems, sems, sems, sems, sems, sems, sems,
        ],
        compiler_params=pltpu.CompilerParams(collective_id=0),
    )(x, w_mat)
